# baseline (speedup 1.0000x reference)
import functools

import jax
import jax.numpy as jnp
from jax import lax
from jax.experimental import pallas as pl
from jax.experimental.pallas import tpu as pltpu
from jax.experimental.pallas import tpu_sc as plsc

_NUM_VOCAB = 100000
_EMBED_DIM = 64
_BATCH = 4096
_HIST = 200
_B = _BATCH * _HIST
_NC, _NS = 2, 16
_NW = _NC * _NS
_B_PER_W = _B // _NW
_GATHER = 128
_G_PER_CHUNK = 4
_CHUNK = _G_PER_CHUNK * _GATHER
_N_CHUNKS = _B_PER_W // _CHUNK


@jax.jit
def _sc_gather(idx2d, table):
    mesh = plsc.VectorSubcoreMesh(core_axis_name="c", subcore_axis_name="s")

    @functools.partial(
        pl.kernel,
        out_type=jax.ShapeDtypeStruct((_B, 2 * _EMBED_DIM), jnp.float32),
        mesh=mesh,
        scratch_types=[
            pltpu.VMEM((_B_PER_W,), jnp.int32),
            pltpu.VMEM((2 * _CHUNK, _EMBED_DIM), jnp.float32),
            pltpu.SemaphoreType.DMA,
            pltpu.SemaphoreType.DMA,
            pltpu.SemaphoreType.DMA,
            pltpu.SemaphoreType.DMA,
        ],
        compiler_params=pltpu.CompilerParams(use_tc_tiling_on_sc=False),
    )
    def k(idx_hbm, table_hbm, out_hbm, idx_v, rows_v, g0, g1, s0, s1):
        gsem = (g0, g1)
        ssem = (s0, s1)
        wid = lax.axis_index("s") * _NC + lax.axis_index("c")
        base = wid * _B_PER_W

        pltpu.sync_copy(idx_hbm.at[pl.ds(wid * _B_PER_W, _B_PER_W)], idx_v)

        def rows_slot(b):
            return rows_v.at[pl.ds(b * _CHUNK, _CHUNK)]

        def fire(ci, b):
            for j in range(_G_PER_CHUNK):
                pltpu.async_copy(
                    table_hbm.at[idx_v.at[pl.ds((ci * _G_PER_CHUNK + j) * _GATHER, _GATHER)]],
                    rows_slot(b).at[pl.ds(j * _GATHER, _GATHER)],
                    gsem[b],
                )

        def store(ci, b):
            pltpu.async_copy(
                rows_slot(b),
                out_hbm.at[pl.ds(base + ci * _CHUNK, _CHUNK), pl.ds(0, _EMBED_DIM)],
                ssem[b],
            )

        def drain_gather(b):
            pltpu.make_async_copy(
                table_hbm.at[pl.ds(0, _CHUNK)],
                rows_slot(b),
                gsem[b],
            ).wait()

        def drain_store(b):
            pltpu.make_async_copy(
                rows_slot(b),
                out_hbm.at[pl.ds(0, _CHUNK), pl.ds(0, _EMBED_DIM)],
                ssem[b],
            ).wait()

        fire(0, 0)

        @pl.loop(0, _N_CHUNKS, step=2)
        def _(ci):
            for b in range(2):
                cur = ci + b
                nxt = cur + 1

                @pl.when(nxt < _N_CHUNKS)
                def _():
                    @pl.when(nxt >= 2)
                    def _():
                        drain_store(1 - b)

                    fire(nxt, 1 - b)

                drain_gather(b)
                store(cur, b)

        drain_store(0)
        drain_store(1)

    return k(idx2d, table)


def _tc_transpose(x128):
    nb = _BATCH // 128

    def body(x_ref, o_ref):
        t = pl.program_id(1)
        x3 = x_ref.reshape(128, _HIST, 2 * _EMBED_DIM)
        o_ref[...] = x3[:, t, : _EMBED_DIM].T

    return pl.pallas_call(
        body,
        grid=(nb, _HIST),
        in_specs=[
            pl.BlockSpec((128 * _HIST, 2 * _EMBED_DIM), lambda i, t: (i, 0)),
        ],
        out_specs=pl.BlockSpec((_EMBED_DIM, 128), lambda i, t: (t, i)),
        out_shape=jax.ShapeDtypeStruct((_HIST * _EMBED_DIM, _BATCH), jnp.float32),
        compiler_params=pltpu.CompilerParams(
            dimension_semantics=("parallel", "arbitrary"),
        ),
    )(x128)


def kernel(seqs, table):
    idx1d = seqs.astype(jnp.int32).reshape(_B)
    out = _sc_gather(idx1d, table)
    out2d = _tc_transpose(out)
    return out2d.reshape(_HIST, _EMBED_DIM, _BATCH).transpose(2, 0, 1)


# speedup vs baseline: 6.4692x; 6.4692x over previous
import functools

import jax
import jax.numpy as jnp
from jax import lax
from jax.experimental import pallas as pl
from jax.experimental.pallas import tpu as pltpu
from jax.experimental.pallas import tpu_sc as plsc

_NUM_VOCAB = 100000
_EMBED_DIM = 64
_BATCH = 4096
_HIST = 200
_B = _BATCH * _HIST
_NC, _NS = 2, 16
_NW = _NC * _NS
_B_PER_W = _B // _NW
_GATHER = 128
_G_PER_CHUNK = 5
_CHUNK = _G_PER_CHUNK * _GATHER
_N_CHUNKS = _B_PER_W // _CHUNK


@jax.jit
def _sc_gather(idx2d, table):
    mesh = plsc.VectorSubcoreMesh(core_axis_name="c", subcore_axis_name="s")

    @functools.partial(
        pl.kernel,
        out_type=jax.ShapeDtypeStruct((_B, 2 * _EMBED_DIM), jnp.float32),
        mesh=mesh,
        scratch_types=[
            pltpu.VMEM((_B_PER_W,), jnp.int32),
            pltpu.VMEM((2 * _CHUNK, _EMBED_DIM), jnp.float32),
            pltpu.SemaphoreType.DMA,
            pltpu.SemaphoreType.DMA,
            pltpu.SemaphoreType.DMA,
            pltpu.SemaphoreType.DMA,
        ],
        compiler_params=pltpu.CompilerParams(use_tc_tiling_on_sc=False),
    )
    def k(idx_hbm, table_hbm, out_hbm, idx_v, rows_v, g0, g1, s0, s1):
        gsem = (g0, g1)
        ssem = (s0, s1)
        wid = lax.axis_index("s") * _NC + lax.axis_index("c")
        base = wid * _B_PER_W

        pltpu.sync_copy(idx_hbm.at[pl.ds(wid * _B_PER_W, _B_PER_W)], idx_v)

        def rows_slot(b):
            return rows_v.at[pl.ds(b * _CHUNK, _CHUNK)]

        def fire(ci, b):
            for j in range(_G_PER_CHUNK):
                pltpu.async_copy(
                    table_hbm.at[idx_v.at[pl.ds((ci * _G_PER_CHUNK + j) * _GATHER, _GATHER)]],
                    rows_slot(b).at[pl.ds(j * _GATHER, _GATHER)],
                    gsem[b],
                )

        def store(ci, b):
            pltpu.async_copy(
                rows_slot(b),
                out_hbm.at[pl.ds(base + ci * _CHUNK, _CHUNK), pl.ds(0, _EMBED_DIM)],
                ssem[b],
            )

        def drain_gather(b):
            pltpu.make_async_copy(
                table_hbm.at[pl.ds(0, _CHUNK)],
                rows_slot(b),
                gsem[b],
            ).wait()

        def drain_store(b):
            pltpu.make_async_copy(
                rows_slot(b),
                out_hbm.at[pl.ds(0, _CHUNK), pl.ds(0, _EMBED_DIM)],
                ssem[b],
            ).wait()

        fire(0, 0)

        @pl.loop(0, _N_CHUNKS, step=2)
        def _(ci):
            for b in range(2):
                cur = ci + b
                nxt = cur + 1

                @pl.when(nxt < _N_CHUNKS)
                def _():
                    @pl.when(nxt >= 2)
                    def _():
                        drain_store(1 - b)

                    fire(nxt, 1 - b)

                drain_gather(b)
                store(cur, b)

        drain_store(0)
        drain_store(1)

    return k(idx2d, table)


def kernel(seqs, table):
    idx1d = seqs.astype(jnp.int32).reshape(_B)
    out = _sc_gather(idx1d, table)
    return out[:, :_EMBED_DIM].reshape(_BATCH, _HIST, _EMBED_DIM)
